# baseline (speedup 1.0000x reference)
import functools

import jax
import jax.numpy as jnp
from jax import lax
from jax.experimental import pallas as pl
from jax.experimental.pallas import tpu as pltpu
from jax.experimental.pallas import tpu_sc as plsc

K = 15
IN_CH = 128
OUT_CH = 128
KP_EXTENT = 1.2
N = 10000
NEIGH = 32

_NW = 32
_TOT = N * NEIGH
_CH = 128
_TPAD = 10016
_NSPLIT = 2
_ROWS = _TOT // _NSPLIT
_NCH = _ROWS // _CH


def _sc_gather_body(part_base,
                    idx_hbm, cx_hbm, cy_hbm, cz_hbm, qx_hbm, qy_hbm, qz_hbm,
                    ftab_hbm,
                    fout_hbm, cout_hbm,
                    idx0, idx1, f0, f1, c0, c1,
                    cx_v, cy_v, cz_v, qx_v, qy_v, qz_v,
                    s_idx0, s_idx1, s_g0, s_g1, s_sf0, s_sf1, s_sc0, s_sc1):
    wid = lax.axis_index("s") * 2 + lax.axis_index("c")

    idxb = (idx0, idx1)
    fb = (f0, f1)
    cb = (c0, c1)
    s_idx = (s_idx0, s_idx1)
    s_g = (s_g0, s_g1)
    s_sf = (s_sf0, s_sf1)
    s_sc = (s_sc0, s_sc1)
    ctabs = (cx_v, cy_v, cz_v)
    qtabs = (qx_v, qy_v, qz_v)

    nfull = _NCH // _NW
    extra = _NCH - nfull * _NW
    nw = nfull + jnp.where(wid < extra, 1, 0)

    def off_of(i):
        return (wid + i * _NW) * _CH

    def prefetch_idx(i, p):
        @pl.when(i < nw)
        def _():
            pltpu.async_copy(idx_hbm.at[pl.ds(off_of(i), _CH)],
                             idxb[p], s_idx[p])

    def wait_idx(p):
        pltpu.make_async_copy(idx_hbm.at[pl.ds(0, _CH)],
                              idxb[p], s_idx[p]).wait()

    def wait_stores(p):
        pltpu.make_async_copy(fb[p], fout_hbm.at[pl.ds(0, _CH)],
                              s_sf[p]).wait()
        pltpu.make_async_copy(cb[p], cout_hbm.at[pl.ds(0, 8), pl.ds(0, _CH)],
                              s_sc[p]).wait()

    def do_chunk(i, p, store_wait):
        off = off_of(i)
        if store_wait:
            wait_stores(p)
        wait_idx(p)
        pltpu.async_copy(ftab_hbm.at[idxb[p]], fb[p], s_g[p])
        prefetch_idx(i + 1, 1 - p)
        for t in range(_CH // 16):
            ivec = idxb[p][pl.ds(t * 16, 16)]
            qvec = (lax.broadcasted_iota(jnp.int32, (16,), 0)
                    + (part_base + off + t * 16)) >> 5
            for j in range(3):
                pc = plsc.load_gather(ctabs[j], [ivec])
                qc = plsc.load_gather(qtabs[j], [qvec])
                cb[p][j, pl.ds(t * 16, 16)] = pc - qc
        pltpu.make_async_copy(ftab_hbm.at[pl.ds(0, _CH)], fb[p],
                              s_g[p]).wait()
        pltpu.async_copy(fb[p], fout_hbm.at[pl.ds(off, _CH)], s_sf[p])
        pltpu.async_copy(cb[p], cout_hbm.at[pl.ds(0, 8), pl.ds(off, _CH)],
                         s_sc[p])

    prefetch_idx(0, 0)
    pltpu.sync_copy(cx_hbm, cx_v)
    pltpu.sync_copy(cy_hbm, cy_v)
    pltpu.sync_copy(cz_hbm, cz_v)
    pltpu.sync_copy(qx_hbm, qx_v)
    pltpu.sync_copy(qy_hbm, qy_v)
    pltpu.sync_copy(qz_hbm, qz_v)

    do_chunk(0, 0, False)
    do_chunk(1, 1, False)

    def body(i, _):
        @pl.when((i & 1) == 0)
        def _():
            do_chunk(i, 0, True)

        @pl.when((i & 1) == 1)
        def _():
            do_chunk(i, 1, True)
        return 0

    lax.fori_loop(2, nw, body, 0)
    wait_stores(0)
    wait_stores(1)


@functools.lru_cache(maxsize=None)
def _get_sc_gather(part_base):
    return pl.kernel(
        functools.partial(_sc_gather_body, part_base),
        out_type=[
            jax.ShapeDtypeStruct((_ROWS, IN_CH), jnp.float32),
            jax.ShapeDtypeStruct((8, _ROWS), jnp.float32),
        ],
        mesh=plsc.VectorSubcoreMesh(core_axis_name="c", subcore_axis_name="s"),
        compiler_params=pltpu.CompilerParams(needs_layout_passes=False),
        scratch_types=(
            [pltpu.VMEM((_CH,), jnp.int32)] * 2
            + [pltpu.VMEM((_CH, IN_CH), jnp.float32)] * 2
            + [pltpu.VMEM((8, _CH), jnp.float32)] * 2
            + [pltpu.VMEM((_TPAD,), jnp.float32)] * 6
            + [pltpu.SemaphoreType.DMA] * 8
        ),
    )


_B = 200
_G = _B // 8
_BH = _B * NEIGH
_NP = N // _NSPLIT


def _tc_body(fg_ref, cg_ref, kpt_ref, mask_ref, w_ref, o_ref, wd_scr, wf_scr):
    r = cg_ref[...]
    d2 = jnp.zeros((16, _BH), jnp.float32)
    for j in range(3):
        rb = jnp.broadcast_to(r[j:j + 1, :], (16, _BH))
        kj = kpt_ref[:, j][:, None]
        t = rb - kj
        d2 = d2 + t * t
    wd_scr[...] = jnp.maximum(1.0 - jnp.sqrt(d2) * (1.0 / KP_EXTENT), 0.0)

    mask = mask_ref[...]
    for g in range(_G):
        wg = wd_scr[:, pl.ds(g * 256, 256)]
        wbd = (jnp.concatenate(
            [jnp.broadcast_to(wg[k:k + 1, :], (8, 256)) for k in range(K)],
            axis=0) * mask).astype(jnp.bfloat16)
        fgg = fg_ref[pl.ds(g * 256, 256), :].astype(jnp.bfloat16)
        wfg = jnp.dot(wbd, fgg, preferred_element_type=jnp.float32)
        for k in range(K):
            wf_scr[k, pl.ds(g * 8, 8), :] = (
                wfg[k * 8:(k + 1) * 8, :].astype(jnp.bfloat16))

    acc = jnp.zeros((_B, OUT_CH), jnp.float32)
    for k in range(K):
        acc = acc + jnp.dot(wf_scr[k], w_ref[k],
                            preferred_element_type=jnp.float32)
    o_ref[...] = acc


def _tc_call(fg, cg, kpt, gmask, wts):
    return pl.pallas_call(
        _tc_body,
        grid=(_NP // _B,),
        in_specs=[
            pl.BlockSpec((_BH, IN_CH), lambda i: (i, 0)),
            pl.BlockSpec((8, _BH), lambda i: (0, i)),
            pl.BlockSpec((16, 8), lambda i: (0, 0)),
            pl.BlockSpec((8 * K, 256), lambda i: (0, 0)),
            pl.BlockSpec((K, IN_CH, OUT_CH), lambda i: (0, 0, 0)),
        ],
        out_specs=pl.BlockSpec((_B, OUT_CH), lambda i: (i, 0)),
        out_shape=jax.ShapeDtypeStruct((_NP, OUT_CH), jnp.float32),
        scratch_shapes=[
            pltpu.VMEM((16, _BH), jnp.float32),
            pltpu.VMEM((K, _B, IN_CH), jnp.bfloat16),
        ],
    )(fg, cg, kpt, gmask, wts)


def kernel(query_points, support_points, neighbors_indices, features, wts,
           kernel_points):
    ftab = jnp.concatenate(
        [features, jnp.zeros((1, IN_CH), jnp.float32)], axis=0)
    cpad = jnp.full((_TPAD - N, 3), 1.0e6, jnp.float32)
    ct = jnp.concatenate([support_points, cpad], axis=0)
    qt = jnp.concatenate([query_points, jnp.zeros((_TPAD - N, 3),
                                                  jnp.float32)], axis=0)

    idx = neighbors_indices.reshape(_TOT)

    kpt = jnp.pad(kernel_points, ((0, 1), (0, 5)))
    gmask = (jnp.tile(
        (lax.broadcasted_iota(jnp.int32, (8, 256), 0)
         == lax.broadcasted_iota(jnp.int32, (8, 256), 1) // 32),
        (K, 1))).astype(jnp.float32)

    outs = []
    for s in range(_NSPLIT):
        fg, cg = _get_sc_gather(s * _ROWS)(
            idx[s * _ROWS:(s + 1) * _ROWS],
            ct[:, 0], ct[:, 1], ct[:, 2], qt[:, 0], qt[:, 1], qt[:, 2], ftab)
        outs.append(_tc_call(fg, cg, kpt, gmask, wts.astype(jnp.bfloat16)))
    return jnp.concatenate(outs, axis=0)


# speedup vs baseline: 1.0150x; 1.0150x over previous
import functools

import jax
import jax.numpy as jnp
from jax import lax
from jax.experimental import pallas as pl
from jax.experimental.pallas import tpu as pltpu
from jax.experimental.pallas import tpu_sc as plsc

K = 15
IN_CH = 128
OUT_CH = 128
KP_EXTENT = 1.2
N = 10000
NEIGH = 32

_NW = 32
_TOT = N * NEIGH
_CH = 128
_TPAD = 10016
_NSPLIT = 1
_ROWS = _TOT // _NSPLIT
_NCH = _ROWS // _CH


def _sc_gather_body(part_base,
                    idx_hbm, cx_hbm, cy_hbm, cz_hbm, qx_hbm, qy_hbm, qz_hbm,
                    ftab_hbm,
                    fout_hbm, cout_hbm,
                    idx0, idx1, f0, f1, c0, c1,
                    cx_v, cy_v, cz_v, qx_v, qy_v, qz_v,
                    s_idx0, s_idx1, s_g0, s_g1, s_sf0, s_sf1, s_sc0, s_sc1):
    wid = lax.axis_index("s") * 2 + lax.axis_index("c")

    idxb = (idx0, idx1)
    fb = (f0, f1)
    cb = (c0, c1)
    s_idx = (s_idx0, s_idx1)
    s_g = (s_g0, s_g1)
    s_sf = (s_sf0, s_sf1)
    s_sc = (s_sc0, s_sc1)
    ctabs = (cx_v, cy_v, cz_v)
    qtabs = (qx_v, qy_v, qz_v)

    nfull = _NCH // _NW
    extra = _NCH - nfull * _NW
    nw = nfull + jnp.where(wid < extra, 1, 0)

    def off_of(i):
        return (wid + i * _NW) * _CH

    def prefetch_idx(i, p):
        @pl.when(i < nw)
        def _():
            pltpu.async_copy(idx_hbm.at[pl.ds(off_of(i), _CH)],
                             idxb[p], s_idx[p])

    def wait_idx(p):
        pltpu.make_async_copy(idx_hbm.at[pl.ds(0, _CH)],
                              idxb[p], s_idx[p]).wait()

    def wait_stores(p):
        pltpu.make_async_copy(fb[p], fout_hbm.at[pl.ds(0, _CH)],
                              s_sf[p]).wait()
        pltpu.make_async_copy(cb[p], cout_hbm.at[pl.ds(0, 8), pl.ds(0, _CH)],
                              s_sc[p]).wait()

    def do_chunk(i, p, store_wait):
        off = off_of(i)
        if store_wait:
            wait_stores(p)
        wait_idx(p)
        pltpu.async_copy(ftab_hbm.at[idxb[p]], fb[p], s_g[p])
        prefetch_idx(i + 1, 1 - p)
        for t in range(_CH // 16):
            ivec = idxb[p][pl.ds(t * 16, 16)]
            qvec = (lax.broadcasted_iota(jnp.int32, (16,), 0)
                    + (part_base + off + t * 16)) >> 5
            for j in range(3):
                pc = plsc.load_gather(ctabs[j], [ivec])
                qc = plsc.load_gather(qtabs[j], [qvec])
                cb[p][j, pl.ds(t * 16, 16)] = pc - qc
        pltpu.make_async_copy(ftab_hbm.at[pl.ds(0, _CH)], fb[p],
                              s_g[p]).wait()
        pltpu.async_copy(fb[p], fout_hbm.at[pl.ds(off, _CH)], s_sf[p])
        pltpu.async_copy(cb[p], cout_hbm.at[pl.ds(0, 8), pl.ds(off, _CH)],
                         s_sc[p])

    prefetch_idx(0, 0)
    pltpu.sync_copy(cx_hbm, cx_v)
    pltpu.sync_copy(cy_hbm, cy_v)
    pltpu.sync_copy(cz_hbm, cz_v)
    pltpu.sync_copy(qx_hbm, qx_v)
    pltpu.sync_copy(qy_hbm, qy_v)
    pltpu.sync_copy(qz_hbm, qz_v)

    do_chunk(0, 0, False)
    do_chunk(1, 1, False)

    def body(i, _):
        @pl.when((i & 1) == 0)
        def _():
            do_chunk(i, 0, True)

        @pl.when((i & 1) == 1)
        def _():
            do_chunk(i, 1, True)
        return 0

    lax.fori_loop(2, nw, body, 0)
    wait_stores(0)
    wait_stores(1)


@functools.lru_cache(maxsize=None)
def _get_sc_gather(part_base):
    return pl.kernel(
        functools.partial(_sc_gather_body, part_base),
        out_type=[
            jax.ShapeDtypeStruct((_ROWS, IN_CH), jnp.float32),
            jax.ShapeDtypeStruct((8, _ROWS), jnp.float32),
        ],
        mesh=plsc.VectorSubcoreMesh(core_axis_name="c", subcore_axis_name="s"),
        compiler_params=pltpu.CompilerParams(needs_layout_passes=False),
        scratch_types=(
            [pltpu.VMEM((_CH,), jnp.int32)] * 2
            + [pltpu.VMEM((_CH, IN_CH), jnp.float32)] * 2
            + [pltpu.VMEM((8, _CH), jnp.float32)] * 2
            + [pltpu.VMEM((_TPAD,), jnp.float32)] * 6
            + [pltpu.SemaphoreType.DMA] * 8
        ),
    )


_B = 200
_G = _B // 8
_BH = _B * NEIGH
_NP = N // _NSPLIT


def _tc_body(fg_ref, cg_ref, kpt_ref, mask_ref, w_ref, o_ref, wd_scr, wf_scr):
    r = cg_ref[...]
    d2 = jnp.zeros((16, _BH), jnp.float32)
    for j in range(3):
        rb = jnp.broadcast_to(r[j:j + 1, :], (16, _BH))
        kj = kpt_ref[:, j][:, None]
        t = rb - kj
        d2 = d2 + t * t
    wd_scr[...] = jnp.maximum(1.0 - jnp.sqrt(d2) * (1.0 / KP_EXTENT), 0.0)

    mask = mask_ref[...]
    for g in range(_G):
        wg = wd_scr[:, pl.ds(g * 256, 256)]
        wbd = jnp.concatenate(
            [jnp.broadcast_to(wg[k:k + 1, :], (8, 256)) for k in range(K)],
            axis=0) * mask
        fgg = fg_ref[pl.ds(g * 256, 256), :]
        wfg = jnp.dot(wbd, fgg, preferred_element_type=jnp.float32)
        for k in range(K):
            wf_scr[k, pl.ds(g * 8, 8), :] = wfg[k * 8:(k + 1) * 8, :]

    acc = jnp.zeros((_B, OUT_CH), jnp.float32)
    for k in range(K):
        acc = acc + jnp.dot(wf_scr[k], w_ref[k],
                            preferred_element_type=jnp.float32)
    o_ref[...] = acc


def _tc_call(fg, cg, kpt, gmask, wts):
    return pl.pallas_call(
        _tc_body,
        grid=(_NP // _B,),
        in_specs=[
            pl.BlockSpec((_BH, IN_CH), lambda i: (i, 0)),
            pl.BlockSpec((8, _BH), lambda i: (0, i)),
            pl.BlockSpec((16, 8), lambda i: (0, 0)),
            pl.BlockSpec((8 * K, 256), lambda i: (0, 0)),
            pl.BlockSpec((K, IN_CH, OUT_CH), lambda i: (0, 0, 0)),
        ],
        out_specs=pl.BlockSpec((_B, OUT_CH), lambda i: (i, 0)),
        out_shape=jax.ShapeDtypeStruct((_NP, OUT_CH), jnp.float32),
        scratch_shapes=[
            pltpu.VMEM((16, _BH), jnp.float32),
            pltpu.VMEM((K, _B, IN_CH), jnp.float32),
        ],
    )(fg, cg, kpt, gmask, wts)


def kernel(query_points, support_points, neighbors_indices, features, wts,
           kernel_points):
    ftab = jnp.concatenate(
        [features, jnp.zeros((1, IN_CH), jnp.float32)], axis=0)
    cpad = jnp.full((_TPAD - N, 3), 1.0e6, jnp.float32)
    ct = jnp.concatenate([support_points, cpad], axis=0)
    qt = jnp.concatenate([query_points, jnp.zeros((_TPAD - N, 3),
                                                  jnp.float32)], axis=0)

    idx = neighbors_indices.reshape(_TOT)

    kpt = jnp.pad(kernel_points, ((0, 1), (0, 5)))
    gmask = (jnp.tile(
        (lax.broadcasted_iota(jnp.int32, (8, 256), 0)
         == lax.broadcasted_iota(jnp.int32, (8, 256), 1) // 32),
        (K, 1))).astype(jnp.float32)

    outs = []
    for s in range(_NSPLIT):
        fg, cg = _get_sc_gather(s * _ROWS)(
            idx[s * _ROWS:(s + 1) * _ROWS],
            ct[:, 0], ct[:, 1], ct[:, 2], qt[:, 0], qt[:, 1], qt[:, 2], ftab)
        outs.append(_tc_call(fg, cg, kpt, gmask, wts))
    return jnp.concatenate(outs, axis=0)


# speedup vs baseline: 1.0392x; 1.0238x over previous
import functools

import jax
import jax.numpy as jnp
from jax import lax
from jax.experimental import pallas as pl
from jax.experimental.pallas import tpu as pltpu
from jax.experimental.pallas import tpu_sc as plsc

K = 15
IN_CH = 128
OUT_CH = 128
KP_EXTENT = 1.2
N = 10000
NEIGH = 32

_NW = 32
_TOT = N * NEIGH
_CH = 128
_TPAD = 10016
_NSPLIT = 1
_ROWS = _TOT // _NSPLIT
_NCH = _ROWS // _CH
assert (_NCH // _NW) % 2 == 0


def _sc_gather_body(part_base,
                    idx_hbm, cx_hbm, cy_hbm, cz_hbm, qx_hbm, qy_hbm, qz_hbm,
                    ftab_hbm,
                    fout_hbm, cout_hbm,
                    idx0, idx1, f0, f1, c0, c1,
                    cx_v, cy_v, cz_v, qx_v, qy_v, qz_v,
                    s_idx0, s_idx1, s_g0, s_g1, s_sf0, s_sf1, s_sc0, s_sc1):
    wid = lax.axis_index("s") * 2 + lax.axis_index("c")

    idxb = (idx0, idx1)
    fb = (f0, f1)
    cb = (c0, c1)
    s_idx = (s_idx0, s_idx1)
    s_g = (s_g0, s_g1)
    s_sf = (s_sf0, s_sf1)
    s_sc = (s_sc0, s_sc1)
    ctabs = (cx_v, cy_v, cz_v)
    qtabs = (qx_v, qy_v, qz_v)

    nfull = _NCH // _NW
    extra = _NCH - nfull * _NW
    nw = nfull + jnp.where(wid < extra, 1, 0)

    def off_of(i):
        return (wid + i * _NW) * _CH

    def prefetch_idx(i, p):
        @pl.when(i < nw)
        def _():
            pltpu.async_copy(idx_hbm.at[pl.ds(off_of(i), _CH)],
                             idxb[p], s_idx[p])

    def wait_idx(p):
        pltpu.make_async_copy(idx_hbm.at[pl.ds(0, _CH)],
                              idxb[p], s_idx[p]).wait()

    def wait_stores(p):
        pltpu.make_async_copy(fb[p], fout_hbm.at[pl.ds(0, _CH)],
                              s_sf[p]).wait()
        pltpu.make_async_copy(cb[p], cout_hbm.at[pl.ds(0, 8), pl.ds(0, _CH)],
                              s_sc[p]).wait()

    def do_chunk(i, p, store_wait):
        off = off_of(i)
        if store_wait:
            wait_stores(p)
        wait_idx(p)
        pltpu.async_copy(ftab_hbm.at[idxb[p]], fb[p], s_g[p])
        prefetch_idx(i + 1, 1 - p)
        for t in range(_CH // 16):
            ivec = idxb[p][pl.ds(t * 16, 16)]
            qvec = (lax.broadcasted_iota(jnp.int32, (16,), 0)
                    + (part_base + off + t * 16)) >> 5
            for j in range(3):
                pc = plsc.load_gather(ctabs[j], [ivec])
                qc = plsc.load_gather(qtabs[j], [qvec])
                cb[p][j, pl.ds(t * 16, 16)] = pc - qc
        pltpu.make_async_copy(ftab_hbm.at[pl.ds(0, _CH)], fb[p],
                              s_g[p]).wait()
        pltpu.async_copy(fb[p], fout_hbm.at[pl.ds(off, _CH)], s_sf[p])
        pltpu.async_copy(cb[p], cout_hbm.at[pl.ds(0, 8), pl.ds(off, _CH)],
                         s_sc[p])

    prefetch_idx(0, 0)
    pltpu.sync_copy(cx_hbm, cx_v)
    pltpu.sync_copy(cy_hbm, cy_v)
    pltpu.sync_copy(cz_hbm, cz_v)
    pltpu.sync_copy(qx_hbm, qx_v)
    pltpu.sync_copy(qy_hbm, qy_v)
    pltpu.sync_copy(qz_hbm, qz_v)

    do_chunk(0, 0, False)
    do_chunk(1, 1, False)

    def body(j, _):
        do_chunk(2 * j, 0, True)
        do_chunk(2 * j + 1, 1, True)
        return 0

    lax.fori_loop(1, nfull // 2, body, 0)

    wait_stores(1)

    @pl.when(nw > nfull)
    def _():
        wait_stores(0)
        off = off_of(nfull)
        wait_idx(0)
        pltpu.async_copy(ftab_hbm.at[idxb[0]], fb[0], s_g[0])
        for t in range(_CH // 16):
            ivec = idxb[0][pl.ds(t * 16, 16)]
            qvec = (lax.broadcasted_iota(jnp.int32, (16,), 0)
                    + (part_base + off + t * 16)) >> 5
            for j in range(3):
                pc = plsc.load_gather(ctabs[j], [ivec])
                qc = plsc.load_gather(qtabs[j], [qvec])
                cb[0][j, pl.ds(t * 16, 16)] = pc - qc
        pltpu.make_async_copy(ftab_hbm.at[pl.ds(0, _CH)], fb[0],
                              s_g[0]).wait()
        pltpu.sync_copy(fb[0], fout_hbm.at[pl.ds(off, _CH)])
        pltpu.sync_copy(cb[0], cout_hbm.at[pl.ds(0, 8), pl.ds(off, _CH)])

    @pl.when(nw == nfull)
    def _():
        wait_stores(0)


@functools.lru_cache(maxsize=None)
def _get_sc_gather(part_base):
    return pl.kernel(
        functools.partial(_sc_gather_body, part_base),
        out_type=[
            jax.ShapeDtypeStruct((_ROWS, IN_CH), jnp.float32),
            jax.ShapeDtypeStruct((8, _ROWS), jnp.float32),
        ],
        mesh=plsc.VectorSubcoreMesh(core_axis_name="c", subcore_axis_name="s"),
        compiler_params=pltpu.CompilerParams(needs_layout_passes=False),
        scratch_types=(
            [pltpu.VMEM((_CH,), jnp.int32)] * 2
            + [pltpu.VMEM((_CH, IN_CH), jnp.float32)] * 2
            + [pltpu.VMEM((8, _CH), jnp.float32)] * 2
            + [pltpu.VMEM((_TPAD,), jnp.float32)] * 6
            + [pltpu.SemaphoreType.DMA] * 8
        ),
    )


_B = 200
_G = _B // 8
_BH = _B * NEIGH
_NP = N // _NSPLIT


def _tc_body(fg_ref, cg_ref, kpt_ref, mask_ref, w_ref, o_ref, wd_scr, wf_scr):
    r = cg_ref[...]
    d2 = jnp.zeros((16, _BH), jnp.float32)
    for j in range(3):
        rb = jnp.broadcast_to(r[j:j + 1, :], (16, _BH))
        kj = kpt_ref[:, j][:, None]
        t = rb - kj
        d2 = d2 + t * t
    wd_scr[...] = jnp.maximum(1.0 - jnp.sqrt(d2) * (1.0 / KP_EXTENT), 0.0)

    mask = mask_ref[...]
    for g in range(_G):
        wg = wd_scr[:, pl.ds(g * 256, 256)]
        wbd = jnp.concatenate(
            [jnp.broadcast_to(wg[k:k + 1, :], (8, 256)) for k in range(K)],
            axis=0) * mask
        fgg = fg_ref[pl.ds(g * 256, 256), :]
        wfg = jnp.dot(wbd, fgg, preferred_element_type=jnp.float32)
        for k in range(K):
            wf_scr[k, pl.ds(g * 8, 8), :] = wfg[k * 8:(k + 1) * 8, :]

    acc = jnp.zeros((_B, OUT_CH), jnp.float32)
    for k in range(K):
        acc = acc + jnp.dot(wf_scr[k], w_ref[k],
                            preferred_element_type=jnp.float32)
    o_ref[...] = acc


def _tc_call(fg, cg, kpt, gmask, wts):
    return pl.pallas_call(
        _tc_body,
        grid=(_NP // _B,),
        in_specs=[
            pl.BlockSpec((_BH, IN_CH), lambda i: (i, 0)),
            pl.BlockSpec((8, _BH), lambda i: (0, i)),
            pl.BlockSpec((16, 8), lambda i: (0, 0)),
            pl.BlockSpec((8 * K, 256), lambda i: (0, 0)),
            pl.BlockSpec((K, IN_CH, OUT_CH), lambda i: (0, 0, 0)),
        ],
        out_specs=pl.BlockSpec((_B, OUT_CH), lambda i: (i, 0)),
        out_shape=jax.ShapeDtypeStruct((_NP, OUT_CH), jnp.float32),
        scratch_shapes=[
            pltpu.VMEM((16, _BH), jnp.float32),
            pltpu.VMEM((K, _B, IN_CH), jnp.float32),
        ],
    )(fg, cg, kpt, gmask, wts)


def kernel(query_points, support_points, neighbors_indices, features, wts,
           kernel_points):
    ftab = jnp.concatenate(
        [features, jnp.zeros((1, IN_CH), jnp.float32)], axis=0)
    cpad = jnp.full((_TPAD - N, 3), 1.0e6, jnp.float32)
    ct = jnp.concatenate([support_points, cpad], axis=0)
    qt = jnp.concatenate([query_points, jnp.zeros((_TPAD - N, 3),
                                                  jnp.float32)], axis=0)

    idx = neighbors_indices.reshape(_TOT)

    kpt = jnp.pad(kernel_points, ((0, 1), (0, 5)))
    gmask = (jnp.tile(
        (lax.broadcasted_iota(jnp.int32, (8, 256), 0)
         == lax.broadcasted_iota(jnp.int32, (8, 256), 1) // 32),
        (K, 1))).astype(jnp.float32)

    outs = []
    for s in range(_NSPLIT):
        fg, cg = _get_sc_gather(s * _ROWS)(
            idx[s * _ROWS:(s + 1) * _ROWS],
            ct[:, 0], ct[:, 1], ct[:, 2], qt[:, 0], qt[:, 1], qt[:, 2], ftab)
        outs.append(_tc_call(fg, cg, kpt, gmask, wts))
    return jnp.concatenate(outs, axis=0)


# speedup vs baseline: 1.1289x; 1.0863x over previous
import functools

import jax
import jax.numpy as jnp
from jax import lax
from jax.experimental import pallas as pl
from jax.experimental.pallas import tpu as pltpu
from jax.experimental.pallas import tpu_sc as plsc

K = 15
IN_CH = 128
OUT_CH = 128
KP_EXTENT = 1.2
N = 10000
NEIGH = 32

_NW = 32
_TOT = N * NEIGH
_CH = 128
_TPAD = 10016
_NSPLIT = 1
_ROWS = _TOT // _NSPLIT
_NCH = _ROWS // _CH
assert (_NCH // _NW) % 2 == 0


def _sc_gather_body(part_base,
                    idx_hbm, cx_hbm, cy_hbm, cz_hbm, qx_hbm, qy_hbm, qz_hbm,
                    ftab_hbm,
                    fout_hbm, cout_hbm,
                    idx0, idx1, f0, f1, c0, c1,
                    cx_v, cy_v, cz_v, qx_v, qy_v, qz_v,
                    s_idx0, s_idx1, s_g0, s_g1, s_sf0, s_sf1, s_sc0, s_sc1):
    wid = lax.axis_index("s") * 2 + lax.axis_index("c")

    idxb = (idx0, idx1)
    fb = (f0, f1)
    cb = (c0, c1)
    s_idx = (s_idx0, s_idx1)
    s_g = (s_g0, s_g1)
    s_sf = (s_sf0, s_sf1)
    s_sc = (s_sc0, s_sc1)
    ctabs = (cx_v, cy_v, cz_v)
    qtabs = (qx_v, qy_v, qz_v)

    nfull = _NCH // _NW
    extra = _NCH - nfull * _NW
    nw = nfull + jnp.where(wid < extra, 1, 0)

    def off_of(i):
        return (wid + i * _NW) * _CH

    def prefetch_idx(i, p):
        @pl.when(i < nw)
        def _():
            pltpu.async_copy(idx_hbm.at[pl.ds(off_of(i), _CH)],
                             idxb[p], s_idx[p])

    def wait_idx(p):
        pltpu.make_async_copy(idx_hbm.at[pl.ds(0, _CH)],
                              idxb[p], s_idx[p]).wait()

    def wait_stores(p):
        pltpu.make_async_copy(fb[p], fout_hbm.at[pl.ds(0, _CH)],
                              s_sf[p]).wait()
        pltpu.make_async_copy(cb[p], cout_hbm.at[pl.ds(0, 8), pl.ds(0, _CH)],
                              s_sc[p]).wait()

    def do_chunk(i, p, store_wait):
        off = off_of(i)
        if store_wait:
            wait_stores(p)
        wait_idx(p)
        pltpu.async_copy(ftab_hbm.at[idxb[p]], fb[p], s_g[p])
        prefetch_idx(i + 1, 1 - p)
        for t in range(_CH // 16):
            ivec = idxb[p][pl.ds(t * 16, 16)]
            qvec = (lax.broadcasted_iota(jnp.int32, (16,), 0)
                    + (part_base + off + t * 16)) >> 5
            for j in range(3):
                pc = plsc.load_gather(ctabs[j], [ivec])
                qc = plsc.load_gather(qtabs[j], [qvec])
                cb[p][j, pl.ds(t * 16, 16)] = pc - qc
        pltpu.make_async_copy(ftab_hbm.at[pl.ds(0, _CH)], fb[p],
                              s_g[p]).wait()
        pltpu.async_copy(fb[p], fout_hbm.at[pl.ds(off, _CH)], s_sf[p])
        pltpu.async_copy(cb[p], cout_hbm.at[pl.ds(0, 8), pl.ds(off, _CH)],
                         s_sc[p])

    prefetch_idx(0, 0)
    pltpu.sync_copy(cx_hbm, cx_v)
    pltpu.sync_copy(cy_hbm, cy_v)
    pltpu.sync_copy(cz_hbm, cz_v)
    pltpu.sync_copy(qx_hbm, qx_v)
    pltpu.sync_copy(qy_hbm, qy_v)
    pltpu.sync_copy(qz_hbm, qz_v)

    do_chunk(0, 0, False)
    do_chunk(1, 1, False)

    def body(j, _):
        do_chunk(2 * j, 0, True)
        do_chunk(2 * j + 1, 1, True)
        return 0

    lax.fori_loop(1, nfull // 2, body, 0)

    wait_stores(1)

    @pl.when(nw > nfull)
    def _():
        wait_stores(0)
        off = off_of(nfull)
        wait_idx(0)
        pltpu.async_copy(ftab_hbm.at[idxb[0]], fb[0], s_g[0])
        for t in range(_CH // 16):
            ivec = idxb[0][pl.ds(t * 16, 16)]
            qvec = (lax.broadcasted_iota(jnp.int32, (16,), 0)
                    + (part_base + off + t * 16)) >> 5
            for j in range(3):
                pc = plsc.load_gather(ctabs[j], [ivec])
                qc = plsc.load_gather(qtabs[j], [qvec])
                cb[0][j, pl.ds(t * 16, 16)] = pc - qc
        pltpu.make_async_copy(ftab_hbm.at[pl.ds(0, _CH)], fb[0],
                              s_g[0]).wait()
        pltpu.sync_copy(fb[0], fout_hbm.at[pl.ds(off, _CH)])
        pltpu.sync_copy(cb[0], cout_hbm.at[pl.ds(0, 8), pl.ds(off, _CH)])

    @pl.when(nw == nfull)
    def _():
        wait_stores(0)


@functools.lru_cache(maxsize=None)
def _get_sc_gather(part_base):
    return pl.kernel(
        functools.partial(_sc_gather_body, part_base),
        out_type=[
            jax.ShapeDtypeStruct((_ROWS, IN_CH), jnp.float32),
            jax.ShapeDtypeStruct((8, _ROWS), jnp.float32),
        ],
        mesh=plsc.VectorSubcoreMesh(core_axis_name="c", subcore_axis_name="s"),
        compiler_params=pltpu.CompilerParams(needs_layout_passes=False),
        scratch_types=(
            [pltpu.VMEM((_CH,), jnp.int32)] * 2
            + [pltpu.VMEM((_CH, IN_CH), jnp.float32)] * 2
            + [pltpu.VMEM((8, _CH), jnp.float32)] * 2
            + [pltpu.VMEM((_TPAD,), jnp.float32)] * 6
            + [pltpu.SemaphoreType.DMA] * 8
        ),
    )


_B = 1000
_G = _B // 8
_BH = _B * NEIGH
_NP = N // _NSPLIT


def _tc_body(fg_ref, cg_ref, kpt_ref, mask_ref, w_ref, o_ref, wd_scr, wf_scr):
    r = cg_ref[...]
    d2 = jnp.zeros((16, _BH), jnp.float32)
    for j in range(3):
        rb = jnp.broadcast_to(r[j:j + 1, :], (16, _BH))
        kj = kpt_ref[:, j][:, None]
        t = rb - kj
        d2 = d2 + t * t
    wd_scr[...] = jnp.maximum(1.0 - jnp.sqrt(d2) * (1.0 / KP_EXTENT), 0.0)

    mask = mask_ref[...]
    for g in range(_G):
        wg = wd_scr[:, pl.ds(g * 256, 256)]
        wbd = jnp.concatenate(
            [jnp.broadcast_to(wg[k:k + 1, :], (8, 256)) for k in range(K)],
            axis=0) * mask
        fgg = fg_ref[pl.ds(g * 256, 256), :]
        wfg = jnp.dot(wbd, fgg, preferred_element_type=jnp.float32)
        for k in range(K):
            wf_scr[k, pl.ds(g * 8, 8), :] = wfg[k * 8:(k + 1) * 8, :]

    acc = jnp.zeros((_B, OUT_CH), jnp.float32)
    for k in range(K):
        acc = acc + jnp.dot(wf_scr[k], w_ref[k],
                            preferred_element_type=jnp.float32)
    o_ref[...] = acc


def _tc_call(fg, cg, kpt, gmask, wts):
    return pl.pallas_call(
        _tc_body,
        grid=(_NP // _B,),
        in_specs=[
            pl.BlockSpec((_BH, IN_CH), lambda i: (i, 0)),
            pl.BlockSpec((8, _BH), lambda i: (0, i)),
            pl.BlockSpec((16, 8), lambda i: (0, 0)),
            pl.BlockSpec((8 * K, 256), lambda i: (0, 0)),
            pl.BlockSpec((K, IN_CH, OUT_CH), lambda i: (0, 0, 0)),
        ],
        out_specs=pl.BlockSpec((_B, OUT_CH), lambda i: (i, 0)),
        out_shape=jax.ShapeDtypeStruct((_NP, OUT_CH), jnp.float32),
        scratch_shapes=[
            pltpu.VMEM((16, _BH), jnp.float32),
            pltpu.VMEM((K, _B, IN_CH), jnp.float32),
        ],
    )(fg, cg, kpt, gmask, wts)


def kernel(query_points, support_points, neighbors_indices, features, wts,
           kernel_points):
    ftab = jnp.concatenate(
        [features, jnp.zeros((1, IN_CH), jnp.float32)], axis=0)
    cpad = jnp.full((_TPAD - N, 3), 1.0e6, jnp.float32)
    ct = jnp.concatenate([support_points, cpad], axis=0)
    qt = jnp.concatenate([query_points, jnp.zeros((_TPAD - N, 3),
                                                  jnp.float32)], axis=0)

    idx = neighbors_indices.reshape(_TOT)

    kpt = jnp.pad(kernel_points, ((0, 1), (0, 5)))
    gmask = (jnp.tile(
        (lax.broadcasted_iota(jnp.int32, (8, 256), 0)
         == lax.broadcasted_iota(jnp.int32, (8, 256), 1) // 32),
        (K, 1))).astype(jnp.float32)

    outs = []
    for s in range(_NSPLIT):
        fg, cg = _get_sc_gather(s * _ROWS)(
            idx[s * _ROWS:(s + 1) * _ROWS],
            ct[:, 0], ct[:, 1], ct[:, 2], qt[:, 0], qt[:, 1], qt[:, 2], ftab)
        outs.append(_tc_call(fg, cg, kpt, gmask, wts))
    return jnp.concatenate(outs, axis=0)
